# baseline (speedup 1.0000x reference)
import functools

import jax
import jax.numpy as jnp
from jax import lax
from jax.experimental import pallas as pl
from jax.experimental.pallas import tpu as pltpu

G0, G1, NH = 480, 360, 32
NPTS = 100000
NCELL = G1 * G1
NCP = 131072
NSEG = G0 * G1
BLK = 2000
NBLK = NPTS // BLK
EPS = 1e-5
NEG = -1e30


def _stats_block(h):
    s = jnp.sum(h, axis=0)[None, None, :]
    sq = jnp.sum(h * h, axis=0)[None, None, :]
    return s, sq


def _p0_kernel(x_ref, s_ref, sq_ref):
    @pl.when(pl.program_id(1) == 0)
    def _():
        s_ref[...] = jnp.zeros_like(s_ref)
        sq_ref[...] = jnp.zeros_like(sq_ref)
    x = x_ref[0]
    s, sq = _stats_block(x)
    s_ref[...] += s
    sq_ref[...] += sq


def _layers(x, args, n):
    sc0, sh0 = args[0][0, 0], args[1][0, 0]
    h = x * sc0 + sh0
    for i in range(1, n + 1):
        W = args[2 + 4 * (i - 1)]
        b = args[3 + 4 * (i - 1)][0, 0]
        h = jnp.dot(h, W, preferred_element_type=jnp.float32) + b
        if i < n:
            sc = args[4 + 4 * (i - 1)][0, 0]
            sh = args[5 + 4 * (i - 1)][0, 0]
            h = jnp.maximum(h * sc + sh, 0.0)
    return h


def _stat_pass_kernel(n, *refs):
    x_ref = refs[0]
    args = [r[...] if r.shape[0] == 1 and len(r.shape) == 3 else r[...]
            for r in refs[1:-2]]
    s_ref, sq_ref = refs[-2], refs[-1]

    @pl.when(pl.program_id(1) == 0)
    def _():
        s_ref[...] = jnp.zeros_like(s_ref)
        sq_ref[...] = jnp.zeros_like(sq_ref)

    h = _layers(x_ref[0], args, n)
    s, sq = _stats_block(h)
    s_ref[...] += s
    sq_ref[...] += sq


def _p4_kernel(*refs):
    x_ref = refs[0]
    args = [r[...] for r in refs[1:-1]]
    out_ref = refs[-1]
    out_ref[0] = _layers(x_ref[0], args, 4)


def _vec(v):
    return v.reshape(1, 1, -1)


def _moment_specs(c):
    blk = pl.BlockSpec((1, 1, c), lambda b, i: (b, 0, 0))
    return [blk, blk]


def _vspec(c):
    return pl.BlockSpec((1, 1, c), lambda b, i: (0, 0, 0))


def _wspec(shape):
    return pl.BlockSpec(shape, lambda b, i: (0,) * len(shape))


def _scale_shift(s, sq, g, b):
    mean = s[0] / NPTS
    var = sq[0] / NPTS - mean * mean
    scale = g * lax.rsqrt(var + EPS)
    shift = b - mean * scale
    return scale, shift


def _mlp(xp, params):
    (bn0_g, bn0_b, W1, b1, bn1_g, bn1_b, W2, b2, bn2_g, bn2_b,
     W3, b3, bn3_g, bn3_b, W4, b4) = params
    grid = (2, NBLK)
    xspec = pl.BlockSpec((1, BLK, 8), lambda b, i: (b, i, 0))
    cp = pltpu.CompilerParams(
        dimension_semantics=("arbitrary", "arbitrary"))
    mom = lambda c: [jax.ShapeDtypeStruct((2, 1, c), jnp.float32)] * 2

    s0, sq0 = pl.pallas_call(
        _p0_kernel, grid=grid,
        in_specs=[xspec], out_specs=_moment_specs(8),
        out_shape=mom(8), compiler_params=cp)(xp)
    results = []
    for b in range(2):
        sc0, sh0 = _scale_shift(s0[b], sq0[b], bn0_g, bn0_b)
        args = [_vec(sc0), _vec(sh0), W1, _vec(b1)]
        specs = [_vspec(8), _vspec(8), _wspec(W1.shape), _vspec(64)]
        dims = [64, 128, 256]
        gs = [bn1_g, bn2_g, bn3_g]
        bs = [bn1_b, bn2_b, bn3_b]
        Ws = [W2, W3, W4]
        lbs = [b2, b3, b4]
        xb = xp[b:b + 1]
        gridb = (1, NBLK)
        for n in (1, 2, 3):
            c = dims[n - 1]
            s, sq = pl.pallas_call(
                functools.partial(_stat_pass_kernel, n), grid=gridb,
                in_specs=[xspec] + specs,
                out_specs=_moment_specs(c),
                out_shape=mom(c)[0:1] + mom(c)[0:1],
                compiler_params=cp)(xb, *args)
            sc, sh = _scale_shift(s[0], sq[0], gs[n - 1], bs[n - 1])
            W, lb = Ws[n - 1], lbs[n - 1]
            args += [_vec(sc), _vec(sh), W, _vec(lb)]
            specs += [_vspec(c), _vspec(c), _wspec(W.shape),
                      _vspec(W.shape[1])]
        h4 = pl.pallas_call(
            _p4_kernel, grid=gridb,
            in_specs=[xspec] + specs,
            out_specs=pl.BlockSpec((1, BLK, 512), lambda b, i: (b, i, 0)),
            out_shape=jax.ShapeDtypeStruct((1, NPTS, 512), jnp.float32),
            compiler_params=cp)(xb, *args)
        results.append(h4)
    return jnp.concatenate(results, axis=0)


CB = 2048


def _final_kernel(g_ref, wc_ref, bc_ref, out_ref):
    g = g_ref[0]
    present = (g[:, 0:1] > NEG)
    gm = jnp.where(present, g, 0.0)
    nhT = lax.dot_general(wc_ref[...], gm, (((0,), (1,)), ((), ())),
                          preferred_element_type=jnp.float32)
    nhT = jnp.maximum(nhT + bc_ref[0, 0][:, None], 0.0)
    out_ref[0] = jnp.where(present.T, nhT, 0.0)


def _final(grid2, Wc, bc):
    nb = NCP // CB
    return pl.pallas_call(
        _final_kernel, grid=(2, nb),
        in_specs=[
            pl.BlockSpec((1, CB, 512), lambda b, i: (b, i, 0)),
            pl.BlockSpec((512, NH), lambda b, i: (0, 0)),
            pl.BlockSpec((1, 1, NH), lambda b, i: (0, 0, 0)),
        ],
        out_specs=pl.BlockSpec((1, NH, CB), lambda b, i: (b, 0, i)),
        out_shape=jax.ShapeDtypeStruct((2, NH, NCP), jnp.float32),
        compiler_params=pltpu.CompilerParams(
            dimension_semantics=("arbitrary", "arbitrary")),
    )(grid2, Wc, _vec(bc))


def kernel(pt_fea, xy_ind, circular_padding, bn0_g, bn0_b, W1, b1, bn1_g,
           bn1_b, W2, b2, bn2_g, bn2_b, W3, b3, bn3_g, bn3_b, W4, b4,
           Wc, bc):
    del circular_padding
    xp = jnp.pad(pt_fea, ((0, 0), (0, 0), (0, 1)))
    W1p = jnp.pad(W1, ((0, 1), (0, 0)))
    g0p = jnp.pad(bn0_g, (0, 1))
    b0p = jnp.pad(bn0_b, (0, 1))
    params = (g0p, b0p, W1p, b1, bn1_g, bn1_b, W2, b2, bn2_g, bn2_b,
              W3, b3, bn3_g, bn3_b, W4, b4)
    h4 = _mlp(xp, params)
    cells = xy_ind[..., 0] * G1 + xy_ind[..., 1]

    grids = []
    for b in range(2):
        seg = jax.ops.segment_max(h4[b], cells[b], num_segments=NCP)
        cnt = jax.ops.segment_sum(jnp.ones((NPTS,), jnp.float32), cells[b],
                                  num_segments=NCP)
        seg = jnp.where((cnt > 0)[:, None], seg, NEG * 2)
        grids.append(seg)
    grid2 = jnp.stack(grids)

    outc = _final(grid2, Wc, bc)[:, :, :NCELL]
    out = jnp.concatenate(
        [outc, jnp.zeros((2, NH, NSEG - NCELL), jnp.float32)], axis=2)
    return out.reshape(2, NH, G0, G1)
